# baseline (speedup 1.0000x reference)
import functools

import jax
import jax.numpy as jnp
from jax import lax
from jax.experimental import pallas as pl
from jax.experimental.pallas import tpu as pltpu
from jax.experimental.pallas import tpu_sc as plsc

F32 = jnp.float32

SC_W = 128
TC_EDGE_BLK = 4000
TC_NODE_BLK = 2000
XPAD = 16
NUM_SC_WORKERS = 32
SUBCORES = 16


def _sig(v):
    return 1.0 / (1.0 + jnp.exp(-v))


def _silu(v):
    return v * _sig(v)


def _dot(a, b):
    return jnp.dot(a, b, preferred_element_type=F32)


def _sc_gather_all(hr_tab, hc_tab, x1d, row2d, col2d):
    n, d = hr_tab.shape
    e = row2d.shape[1]
    steps = e // SC_W
    mesh = plsc.VectorSubcoreMesh(core_axis_name="c", subcore_axis_name="s")

    @functools.partial(
        pl.kernel,
        out_type=(
            jax.ShapeDtypeStruct((e, d), F32),
            jax.ShapeDtypeStruct((e, d), F32),
            jax.ShapeDtypeStruct((e, XPAD), F32),
        ),
        mesh=mesh,
        scratch_types=[
            pltpu.SemaphoreType.DMA,
            pltpu.SemaphoreType.DMA,
            pltpu.VMEM((n * 3,), F32),
        ],
        compiler_params=pltpu.CompilerParams(needs_layout_passes=False),
    )
    def k(hr_hbm, hc_hbm, x_hbm, row_hbm, col_hbm,
          hr_o_hbm, hc_o_hbm, rel_o_hbm, s0, s1, x_v):
        pltpu.sync_copy(x_hbm, x_v)
        lanes = jax.lax.iota(jnp.int32, 16)
        zeros16 = jnp.zeros((16,), F32)

        def body(ri, ci, hr_o, hc_o, rel_o):
            c0 = pltpu.async_copy(hr_hbm.at[ri.at[0]], hr_o, s0)
            c1 = pltpu.async_copy(hc_hbm.at[ci.at[0]], hc_o, s1)

            @pl.loop(0, SC_W // 16)
            def _(g):
                base = g * 16
                iv = ri[0, pl.ds(base, 16)] * 3
                cv = ci[0, pl.ds(base, 16)] * 3
                rows = lanes + base
                for c in range(3):
                    xr = plsc.load_gather(x_v, [iv + c])
                    xc = plsc.load_gather(x_v, [cv + c])
                    plsc.store_scatter(rel_o, [rows, jnp.full((16,), c, jnp.int32)],
                                       xr - xc)
                plsc.store_scatter(rel_o, [rows, jnp.full((16,), 3, jnp.int32)],
                                   zeros16)

            c0.wait()
            c1.wait()

        pltpu.emit_pipeline(
            body,
            grid=(steps,),
            in_specs=[
                pl.BlockSpec((1, SC_W), lambda i: (0, i)),
                pl.BlockSpec((1, SC_W), lambda i: (0, i)),
            ],
            out_specs=[
                pl.BlockSpec((SC_W, d), lambda i: (i, 0)),
                pl.BlockSpec((SC_W, d), lambda i: (i, 0)),
                pl.BlockSpec((SC_W, XPAD), lambda i: (i, 0)),
            ],
            core_axis_name=("c", "s"),
            dimension_semantics=(pltpu.PARALLEL,),
        )(row_hbm, col_hbm, hr_o_hbm, hc_o_hbm, rel_o_hbm)

    return k(hr_tab, hc_tab, x1d, row2d, col2d)


def _sc_scatter(data, row1d, init2, untiled):
    e, w = data.shape
    n = init2.shape[1]
    steps = e // SC_W
    rpw = (n // (8 * SUBCORES)) * 8
    tail = n - SUBCORES * rpw
    mesh = plsc.VectorSubcoreMesh(core_axis_name="c", subcore_axis_name="s")
    cp = pltpu.CompilerParams(use_tc_tiling_on_sc=False) if untiled else None

    @functools.partial(
        pl.kernel,
        out_type=jax.ShapeDtypeStruct((2, n, w), F32),
        mesh=mesh,
        scratch_types=(
            pltpu.VMEM_SHARED((n, w), F32),
            pltpu.VMEM((2, SC_W, w), F32),
            pltpu.VMEM((2, SC_W), jnp.int32),
            pltpu.SemaphoreType.DMA,
            pltpu.SemaphoreType.DMA,
            pltpu.SemaphoreType.DMA,
            pltpu.SemaphoreType.DMA,
        ),
        compiler_params=cp,
    )
    def k(d_hbm, row_hbm, z_hbm, p_hbm, acc, d_v, idx_v, sd0, sd1, si0, si1):
        cid = lax.axis_index("c")
        sid = lax.axis_index("s")
        wid = cid * SUBCORES + sid
        sl = pl.ds(sid * rpw, rpw)
        tl = pl.ds(SUBCORES * rpw, tail)
        pltpu.sync_copy(z_hbm.at[cid, sl], acc.at[sl])
        if tail:
            @pl.when(sid == SUBCORES - 1)
            def _():
                pltpu.sync_copy(z_hbm.at[cid, tl], acc.at[tl])

        sems = ((si0, sd0), (si1, sd1))

        def _load(t, b):
            base = t * SC_W
            pltpu.async_copy(row_hbm.at[pl.ds(base, SC_W)], idx_v.at[b],
                             sems[b][0])
            pltpu.async_copy(d_hbm.at[pl.ds(base, SC_W)], d_v.at[b],
                             sems[b][1])

        def _wait(t, b):
            base = t * SC_W
            pltpu.make_async_copy(row_hbm.at[pl.ds(base, SC_W)], idx_v.at[b],
                                  sems[b][0]).wait()
            pltpu.make_async_copy(d_hbm.at[pl.ds(base, SC_W)], d_v.at[b],
                                  sems[b][1]).wait()

        my_steps = (steps - 1 - wid) // NUM_SC_WORKERS + 1
        for b in (0, 1):
            @pl.when(b < my_steps)
            def _(b=b):
                _load(wid + b * NUM_SC_WORKERS, b)
        plsc.subcore_barrier()

        @pl.loop(0, (steps // NUM_SC_WORKERS) + 2, step=2)
        def _(j0):
            for b in (0, 1):
                j = j0 + b

                @pl.when(j < my_steps)
                def _(j=j, b=b):
                    t = wid + j * NUM_SC_WORKERS
                    _wait(t, b)
                    pltpu.sync_copy(d_v.at[b], acc.at[idx_v.at[b]], add=True)

                    @pl.when(j + 2 < my_steps)
                    def _(j=j, b=b):
                        _load(wid + (j + 2) * NUM_SC_WORKERS, b)

        plsc.subcore_barrier()
        pltpu.sync_copy(acc.at[sl], p_hbm.at[cid, sl])
        if tail:
            @pl.when(sid == SUBCORES - 1)
            def _():
                pltpu.sync_copy(acc.at[tl], p_hbm.at[cid, tl])

    return k(data, row1d, init2)


def _tc_project(h, w1r, b1, w1c):
    n, d = h.shape
    grid = (n // TC_NODE_BLK,)
    blk = lambda r, c: pl.BlockSpec((r, c), lambda i: (i, 0))
    full = lambda r, c: pl.BlockSpec((r, c), lambda i: (0, 0))

    def body(h_ref, w1r_ref, b1_ref, w1c_ref, hr_ref, hc_ref):
        hb = h_ref[...]
        hr_ref[...] = _dot(hb, w1r_ref[...]) + b1_ref[...]
        hc_ref[...] = _dot(hb, w1c_ref[...])

    return pl.pallas_call(
        body,
        grid=grid,
        in_specs=[blk(TC_NODE_BLK, d), full(d, d), full(1, d), full(d, d)],
        out_specs=[blk(TC_NODE_BLK, d), blk(TC_NODE_BLK, d)],
        out_shape=[jax.ShapeDtypeStruct((n, d), F32)] * 2,
    )(h, w1r, b1, w1c)


def _tc_edge(hr_g, hc_g, rel_flat, ea, wts, with_coord):
    e, d = hr_g.shape
    grid = (e // TC_EDGE_BLK,)
    ed = ea.shape[1]
    blk = lambda r, c: pl.BlockSpec((r, c), lambda i: (i, 0))
    full = lambda r, c: pl.BlockSpec((r, c), lambda i: (0, 0))

    (wea, w1d, w2, b2, w3, b3, wa1, ba1, wa2r, ba2,
     wc1, bc1, wc2, bc2, wc3p, bc3p) = wts

    def body(hr_ref, hc_ref, rel_ref, ea_ref,
             wea_ref, w1d_ref, w2_ref, b2_ref, w3_ref, b3_ref,
             wa1_ref, ba1_ref, wa2r_ref, ba2_ref,
             wc1_ref, bc1_ref, wc2_ref, bc2_ref, wc3p_ref, bc3p_ref,
             eo_ref, cw_ref=None):
        r4 = rel_ref[:, :4]
        dist = jnp.sqrt(jnp.sum(r4 * r4, axis=-1, keepdims=True))
        z1 = (hr_ref[...] + hc_ref[...]
              + _dot(ea_ref[...], wea_ref[...])
              + dist * w1d_ref[...])
        v = _silu(z1)
        v = _silu(_dot(v, w2_ref[...]) + b2_ref[...])
        e3 = _dot(v, w3_ref[...]) + b3_ref[...]
        t = _silu(_dot(e3, wa1_ref[...]) + ba1_ref[...])
        logit = jnp.sum(t * wa2r_ref[...], axis=-1, keepdims=True) + ba2_ref[:, :1]
        eo = e3 * _sig(logit)
        eo_ref[...] = eo
        if cw_ref is not None:
            c = _silu(_dot(eo, wc1_ref[...]) + bc1_ref[...])
            c = _silu(_dot(c, wc2_ref[...]) + bc2_ref[...])
            c = _dot(c, wc3p_ref[...]) + bc3p_ref[...]
            r16 = jnp.concatenate(
                [r4, jnp.zeros((r4.shape[0], XPAD - 4), F32)], axis=-1)
            cw_ref[...] = c * r16 / (dist + 1e-8)

    in_specs = [
        blk(TC_EDGE_BLK, d), blk(TC_EDGE_BLK, d),
        blk(TC_EDGE_BLK, XPAD), blk(TC_EDGE_BLK, ed),
        full(ed, d), full(1, d), full(d, d), full(1, d), full(d, d), full(1, d),
        full(d, d), full(1, d), full(1, d), full(1, d),
        full(d, d), full(1, d), full(d, d), full(1, d), full(d, XPAD), full(1, XPAD),
    ]
    out_specs = [blk(TC_EDGE_BLK, d)]
    out_shape = [jax.ShapeDtypeStruct((e, d), F32)]
    if with_coord:
        out_specs.append(blk(TC_EDGE_BLK, XPAD))
        out_shape.append(jax.ShapeDtypeStruct((e, XPAD), F32))

    res = pl.pallas_call(
        body, grid=grid, in_specs=in_specs, out_specs=out_specs,
        out_shape=out_shape,
    )(hr_g, hc_g, rel_flat, ea,
      wea, w1d, w2, b2, w3, b3, wa1, ba1, wa2r, ba2,
      wc1, bc1, wc2, bc2, wc3p, bc3p)
    if with_coord:
        return res
    return res[0], None


def _tc_node(h, xpad, pd, px, nwts, lnorm, nxt, fin):
    n, d = h.shape
    grid = (n // TC_NODE_BLK,)
    blk = lambda r, c: pl.BlockSpec((r, c), lambda i: (i, 0))
    blk3 = lambda r, c: pl.BlockSpec((1, r, c), lambda i: (0, i, 0))
    full = lambda r, c: pl.BlockSpec((r, c), lambda i: (0, 0))

    wn1h, wn1a, bn1, wn2, bn2, wn3, bn3 = nwts
    with_coord = px is not None

    def body(*refs):
        it = iter(refs)
        h_ref = next(it)
        pd0_ref, pd1_ref = next(it), next(it)
        if with_coord:
            xp_ref, px0_ref, px1_ref = next(it), next(it), next(it)
        (wn1h_ref, wn1a_ref, bn1_ref, wn2_ref, bn2_ref,
         wn3_ref, bn3_ref) = (next(it) for _ in range(7))
        if lnorm is not None:
            g_ref, gb_ref = next(it), next(it)
        if fin is not None:
            wo_ref, bo_ref = next(it), next(it)
        else:
            w1r_ref, b1_ref, w1c_ref = next(it), next(it), next(it)
        h_out_ref = next(it)
        if with_coord:
            xp_out_ref = next(it)
        if fin is not None:
            fin_ref = next(it)
        else:
            hr_ref, hc_ref = next(it), next(it)

        hb = h_ref[...]
        aggr = pd0_ref[0] + pd1_ref[0]
        z = (_dot(hb, wn1h_ref[...])
             + _dot(aggr, wn1a_ref[...])
             + bn1_ref[...])
        u = _silu(z)
        u = _silu(_dot(u, wn2_ref[...]) + bn2_ref[...])
        mlp = _dot(u, wn3_ref[...]) + bn3_ref[...]
        h_new = hb + mlp
        if lnorm is not None:
            mu = jnp.mean(h_new, axis=-1, keepdims=True)
            var = jnp.mean((h_new - mu) ** 2, axis=-1, keepdims=True)
            h_new = (h_new - mu) / jnp.sqrt(var + 1e-5) * g_ref[...] + gb_ref[...]
            h_new = hb + h_new
        h_out_ref[...] = h_new
        if with_coord:
            xp_out_ref[...] = xp_ref[...] + px0_ref[0] + px1_ref[0]
        if fin is not None:
            fin_ref[...] = _dot(h_new, wo_ref[...]) + bo_ref[...]
        else:
            hr_ref[...] = _dot(h_new, w1r_ref[...]) + b1_ref[...]
            hc_ref[...] = _dot(h_new, w1c_ref[...])

    in_specs = [blk(TC_NODE_BLK, d), blk3(TC_NODE_BLK, d), blk3(TC_NODE_BLK, d)]
    args = [h, pd[:1], pd[1:]]
    if with_coord:
        in_specs += [blk(TC_NODE_BLK, XPAD), blk3(TC_NODE_BLK, XPAD),
                     blk3(TC_NODE_BLK, XPAD)]
        args += [xpad, px[:1], px[1:]]
    in_specs += [full(d, d), full(d, d), full(1, d), full(d, d), full(1, d),
                 full(d, d), full(1, d)]
    args += list(nwts)
    if lnorm is not None:
        in_specs += [full(1, d), full(1, d)]
        args += list(lnorm)
    if fin is not None:
        od = fin[0].shape[1]
        in_specs += [full(d, od), full(1, od)]
        args += list(fin)
    else:
        in_specs += [full(d, d), full(1, d), full(d, d)]
        args += list(nxt)

    out_specs = [blk(TC_NODE_BLK, d)]
    out_shape = [jax.ShapeDtypeStruct((n, d), F32)]
    if with_coord:
        out_specs.append(blk(TC_NODE_BLK, XPAD))
        out_shape.append(jax.ShapeDtypeStruct((n, XPAD), F32))
    if fin is not None:
        od = fin[0].shape[1]
        out_specs.append(blk(TC_NODE_BLK, od))
        out_shape.append(jax.ShapeDtypeStruct((n, od), F32))
    else:
        out_specs += [blk(TC_NODE_BLK, d), blk(TC_NODE_BLK, d)]
        out_shape += [jax.ShapeDtypeStruct((n, d), F32)] * 2

    res = pl.pallas_call(
        body, grid=grid, in_specs=in_specs, out_specs=out_specs,
        out_shape=out_shape,
    )(*args)

    it = iter(res)
    h_out = next(it)
    xp_out = next(it) if with_coord else None
    rest = tuple(it)
    return h_out, xp_out, rest


def _edge_weights(p, d, ed):
    (w1, b1), (w2, b2), (w3, b3) = p["edge"]
    wea = w1[:ed]
    w1r = w1[ed:ed + d]
    w1c = w1[ed + d:ed + 2 * d]
    w1d = w1[ed + 2 * d:ed + 2 * d + 1]
    (wa1, ba1), (wa2, ba2) = p["att"]
    (wc1, bc1), (wc2, bc2), (wc3, bc3) = p["coord"]
    cd = wc3.shape[1]
    wc3p = jnp.pad(wc3, ((0, 0), (0, XPAD - cd)))
    bc3p = jnp.pad(bc3.reshape(1, cd), ((0, 0), (0, XPAD - cd)))
    r2 = lambda b: b.reshape(1, -1)
    wts = (wea, w1d, w2, r2(b2), w3, r2(b3), wa1, r2(ba1),
           wa2.reshape(1, d), jnp.broadcast_to(ba2.reshape(1, 1), (1, d)),
           wc1, r2(bc1), wc2, r2(bc2), wc3p, bc3p)
    return wts, (w1r, r2(b1), w1c)


def kernel(h, x, edge_index, edge_attr, params):
    n, d = h.shape
    cd = x.shape[1]
    e = edge_index.shape[1]
    row = edge_index[0].astype(jnp.int32)
    col = edge_index[1].astype(jnp.int32)
    row2d = row.reshape(1, e)
    col2d = col.reshape(1, e)
    xpad = jnp.pad(x.astype(F32), ((0, 0), (0, XPAD - cd)))
    z_d = jnp.zeros((2, n, d), F32)
    z_x = jnp.zeros((2, n, XPAD), F32)

    layers = params["layers"]
    nlayers = len(layers)
    ed = edge_attr.shape[1]
    ewts = []
    projs = []
    for p in layers:
        w, pr = _edge_weights(p, d, ed)
        ewts.append(w)
        projs.append(pr)

    hr_tab, hc_tab = _tc_project(h, *projs[0])
    out = None
    half = e // 2
    rows_h = [row[:half], row[half:]]
    row2d_h = [row2d[:, :half], row2d[:, half:]]
    col2d_h = [col2d[:, :half], col2d[:, half:]]
    ea_h = [edge_attr[:half], edge_attr[half:]]
    for l, p in enumerate(layers):
        last = l == nlayers - 1
        x1d = xpad[:, :3].reshape(-1)
        pd, px = z_d, z_x
        for ch in (0, 1):
            hr_g, hc_g, rel_flat = _sc_gather_all(
                hr_tab, hc_tab, x1d, row2d_h[ch], col2d_h[ch])
            eo, cw = _tc_edge(hr_g, hc_g, rel_flat, ea_h[ch], ewts[l],
                              with_coord=not last)
            pd = _sc_scatter(eo, rows_h[ch], pd, untiled=False)
            if not last:
                _, cw = jax.lax.optimization_barrier((pd, cw))
                px = _sc_scatter(cw, rows_h[ch], px, untiled=True)
        (wn1, bn1), (wn2, bn2), (wn3, bn3) = p["node"]
        nwts = (wn1[:d], wn1[d:], bn1.reshape(1, -1), wn2,
                bn2.reshape(1, -1), wn3, bn3.reshape(1, -1))
        lnorm = None
        if l > 0:
            g, gb = p["ln"]
            lnorm = (g.reshape(1, -1), gb.reshape(1, -1))
        if last:
            wo, bo = params["out"]
            fin = (wo, bo.reshape(1, -1))
            h, xpad, rest = _tc_node(h, None, pd, None, nwts, lnorm, None, fin)
            out = rest[0]
        else:
            h, xpad, rest = _tc_node(h, xpad, pd, px, nwts, lnorm,
                                     projs[l + 1], None)
            hr_tab, hc_tab = rest
    return out


# speedup vs baseline: 1.3086x; 1.3086x over previous
import functools

import jax
import jax.numpy as jnp
from jax import lax
from jax.experimental import pallas as pl
from jax.experimental.pallas import tpu as pltpu
from jax.experimental.pallas import tpu_sc as plsc

F32 = jnp.float32

SC_W = 128
TC_EDGE_BLK = 2000
TC_NODE_BLK = 2000
XPAD = 16
NUM_SC_WORKERS = 32
SUBCORES = 16


def _sig(v):
    return 1.0 / (1.0 + jnp.exp(-v))


def _silu(v):
    return v * _sig(v)


def _dot(a, b):
    return jnp.dot(a, b, preferred_element_type=F32)


def _sc_gather_pair(tab_a, tab_b, row2d, col2d, untiled):
    n, w = tab_a.shape
    e = row2d.shape[1]
    steps = e // SC_W
    mesh = plsc.VectorSubcoreMesh(core_axis_name="c", subcore_axis_name="s")
    cp = pltpu.CompilerParams(use_tc_tiling_on_sc=False) if untiled else None

    @functools.partial(
        pl.kernel,
        out_type=(
            jax.ShapeDtypeStruct((e, w), F32),
            jax.ShapeDtypeStruct((e, w), F32),
        ),
        mesh=mesh,
        scratch_types=[pltpu.SemaphoreType.DMA] * 2,
        compiler_params=cp,
    )
    def k(a_hbm, b_hbm, row_hbm, col_hbm, a_o_hbm, b_o_hbm, s0, s1):
        def body(ri, ci, a_o, b_o):
            c0 = pltpu.async_copy(a_hbm.at[ri.at[0]], a_o, s0)
            c1 = pltpu.async_copy(b_hbm.at[ci.at[0]], b_o, s1)
            c0.wait()
            c1.wait()

        pltpu.emit_pipeline(
            body,
            grid=(steps,),
            in_specs=[
                pl.BlockSpec((1, SC_W), lambda i: (0, i)),
                pl.BlockSpec((1, SC_W), lambda i: (0, i)),
            ],
            out_specs=[
                pl.BlockSpec((SC_W, w), lambda i: (i, 0)),
                pl.BlockSpec((SC_W, w), lambda i: (i, 0)),
            ],
            core_axis_name=("c", "s"),
            dimension_semantics=(pltpu.PARALLEL,),
        )(row_hbm, col_hbm, a_o_hbm, b_o_hbm)

    return k(tab_a, tab_b, row2d, col2d)


def _sc_scatter(data, row1d, init2, untiled):
    e, w = data.shape
    n = init2.shape[1]
    steps = e // SC_W
    rpw = (n // (8 * SUBCORES)) * 8
    tail = n - SUBCORES * rpw
    mesh = plsc.VectorSubcoreMesh(core_axis_name="c", subcore_axis_name="s")
    cp = pltpu.CompilerParams(use_tc_tiling_on_sc=False) if untiled else None

    @functools.partial(
        pl.kernel,
        out_type=jax.ShapeDtypeStruct((2, n, w), F32),
        mesh=mesh,
        scratch_types=(
            pltpu.VMEM_SHARED((n, w), F32),
            pltpu.VMEM((2, SC_W, w), F32),
            pltpu.VMEM((2, SC_W), jnp.int32),
            pltpu.SemaphoreType.DMA,
            pltpu.SemaphoreType.DMA,
            pltpu.SemaphoreType.DMA,
            pltpu.SemaphoreType.DMA,
        ),
        compiler_params=cp,
    )
    def k(d_hbm, row_hbm, z_hbm, p_hbm, acc, d_v, idx_v, sd0, sd1, si0, si1):
        cid = lax.axis_index("c")
        sid = lax.axis_index("s")
        wid = cid * SUBCORES + sid
        sl = pl.ds(sid * rpw, rpw)
        tl = pl.ds(SUBCORES * rpw, tail)
        pltpu.sync_copy(z_hbm.at[cid, sl], acc.at[sl])
        if tail:
            @pl.when(sid == SUBCORES - 1)
            def _():
                pltpu.sync_copy(z_hbm.at[cid, tl], acc.at[tl])

        sems = ((si0, sd0), (si1, sd1))

        def _load(t, b):
            base = t * SC_W
            pltpu.async_copy(row_hbm.at[pl.ds(base, SC_W)], idx_v.at[b],
                             sems[b][0])
            pltpu.async_copy(d_hbm.at[pl.ds(base, SC_W)], d_v.at[b],
                             sems[b][1])

        def _wait(t, b):
            base = t * SC_W
            pltpu.make_async_copy(row_hbm.at[pl.ds(base, SC_W)], idx_v.at[b],
                                  sems[b][0]).wait()
            pltpu.make_async_copy(d_hbm.at[pl.ds(base, SC_W)], d_v.at[b],
                                  sems[b][1]).wait()

        my_steps = (steps - 1 - wid) // NUM_SC_WORKERS + 1
        for b in (0, 1):
            @pl.when(b < my_steps)
            def _(b=b):
                _load(wid + b * NUM_SC_WORKERS, b)
        plsc.subcore_barrier()

        @pl.loop(0, (steps // NUM_SC_WORKERS) + 2, step=2)
        def _(j0):
            for b in (0, 1):
                j = j0 + b

                @pl.when(j < my_steps)
                def _(j=j, b=b):
                    t = wid + j * NUM_SC_WORKERS
                    _wait(t, b)
                    pltpu.sync_copy(d_v.at[b], acc.at[idx_v.at[b]], add=True)

                    @pl.when(j + 2 < my_steps)
                    def _(j=j, b=b):
                        _load(wid + (j + 2) * NUM_SC_WORKERS, b)

        plsc.subcore_barrier()
        pltpu.sync_copy(acc.at[sl], p_hbm.at[cid, sl])
        if tail:
            @pl.when(sid == SUBCORES - 1)
            def _():
                pltpu.sync_copy(acc.at[tl], p_hbm.at[cid, tl])

    return k(data, row1d, init2)


def _tc_project(h, w1r, b1, w1c):
    n, d = h.shape
    grid = (n // TC_NODE_BLK,)
    blk = lambda r, c: pl.BlockSpec((r, c), lambda i: (i, 0))
    full = lambda r, c: pl.BlockSpec((r, c), lambda i: (0, 0))

    def body(h_ref, w1r_ref, b1_ref, w1c_ref, hr_ref, hc_ref):
        hb = h_ref[...]
        hr_ref[...] = _dot(hb, w1r_ref[...]) + b1_ref[...]
        hc_ref[...] = _dot(hb, w1c_ref[...])

    return pl.pallas_call(
        body,
        grid=grid,
        in_specs=[blk(TC_NODE_BLK, d), full(d, d), full(1, d), full(d, d)],
        out_specs=[blk(TC_NODE_BLK, d), blk(TC_NODE_BLK, d)],
        out_shape=[jax.ShapeDtypeStruct((n, d), F32)] * 2,
    )(h, w1r, b1, w1c)


def _tc_edge(hr_g, hc_g, xr_g, xc_g, ea, wts, with_coord):
    e, d = hr_g.shape
    grid = (e // TC_EDGE_BLK,)
    ed = ea.shape[1]
    blk = lambda r, c: pl.BlockSpec((r, c), lambda i: (i, 0))
    full = lambda r, c: pl.BlockSpec((r, c), lambda i: (0, 0))

    (wea, w1d, w2, b2, w3, b3, wa1, ba1, wa2r, ba2,
     wc1, bc1, wc2, bc2, wc3p, bc3p) = wts

    def body(hr_ref, hc_ref, xr_ref, xc_ref, ea_ref,
             wea_ref, w1d_ref, w2_ref, b2_ref, w3_ref, b3_ref,
             wa1_ref, ba1_ref, wa2r_ref, ba2_ref,
             wc1_ref, bc1_ref, wc2_ref, bc2_ref, wc3p_ref, bc3p_ref,
             eo_ref, cw_ref=None):
        r = xr_ref[...] - xc_ref[...]
        dist = jnp.sqrt(jnp.sum(r * r, axis=-1, keepdims=True))
        z1 = (hr_ref[...] + hc_ref[...]
              + _dot(ea_ref[...], wea_ref[...])
              + dist * w1d_ref[...])
        v = _silu(z1)
        v = _silu(_dot(v, w2_ref[...]) + b2_ref[...])
        e3 = _dot(v, w3_ref[...]) + b3_ref[...]
        t = _silu(_dot(e3, wa1_ref[...]) + ba1_ref[...])
        logit = jnp.sum(t * wa2r_ref[...], axis=-1, keepdims=True) + ba2_ref[:, :1]
        eo = e3 * _sig(logit)
        eo_ref[...] = eo
        if cw_ref is not None:
            c = _silu(_dot(eo, wc1_ref[...]) + bc1_ref[...])
            c = _silu(_dot(c, wc2_ref[...]) + bc2_ref[...])
            c = _dot(c, wc3p_ref[...]) + bc3p_ref[...]
            cw_ref[...] = c * r / (dist + 1e-8)

    in_specs = [
        blk(TC_EDGE_BLK, d), blk(TC_EDGE_BLK, d),
        blk(TC_EDGE_BLK, XPAD), blk(TC_EDGE_BLK, XPAD), blk(TC_EDGE_BLK, ed),
        full(ed, d), full(1, d), full(d, d), full(1, d), full(d, d), full(1, d),
        full(d, d), full(1, d), full(1, d), full(1, d),
        full(d, d), full(1, d), full(d, d), full(1, d), full(d, XPAD), full(1, XPAD),
    ]
    out_specs = [blk(TC_EDGE_BLK, d)]
    out_shape = [jax.ShapeDtypeStruct((e, d), F32)]
    if with_coord:
        out_specs.append(blk(TC_EDGE_BLK, XPAD))
        out_shape.append(jax.ShapeDtypeStruct((e, XPAD), F32))

    res = pl.pallas_call(
        body, grid=grid, in_specs=in_specs, out_specs=out_specs,
        out_shape=out_shape,
    )(hr_g, hc_g, xr_g, xc_g, ea,
      wea, w1d, w2, b2, w3, b3, wa1, ba1, wa2r, ba2,
      wc1, bc1, wc2, bc2, wc3p, bc3p)
    if with_coord:
        return res
    return res[0], None


def _tc_node(h, xpad, pd, px, nwts, lnorm, nxt, fin):
    n, d = h.shape
    grid = (n // TC_NODE_BLK,)
    blk = lambda r, c: pl.BlockSpec((r, c), lambda i: (i, 0))
    blk3 = lambda r, c: pl.BlockSpec((1, r, c), lambda i: (0, i, 0))
    full = lambda r, c: pl.BlockSpec((r, c), lambda i: (0, 0))

    wn1h, wn1a, bn1, wn2, bn2, wn3, bn3 = nwts
    with_coord = px is not None

    def body(*refs):
        it = iter(refs)
        h_ref = next(it)
        pd0_ref, pd1_ref = next(it), next(it)
        if with_coord:
            xp_ref, px0_ref, px1_ref = next(it), next(it), next(it)
        (wn1h_ref, wn1a_ref, bn1_ref, wn2_ref, bn2_ref,
         wn3_ref, bn3_ref) = (next(it) for _ in range(7))
        if lnorm is not None:
            g_ref, gb_ref = next(it), next(it)
        if fin is not None:
            wo_ref, bo_ref = next(it), next(it)
        else:
            w1r_ref, b1_ref, w1c_ref = next(it), next(it), next(it)
        h_out_ref = next(it)
        if with_coord:
            xp_out_ref = next(it)
        if fin is not None:
            fin_ref = next(it)
        else:
            hr_ref, hc_ref = next(it), next(it)

        hb = h_ref[...]
        aggr = pd0_ref[0] + pd1_ref[0]
        z = (_dot(hb, wn1h_ref[...])
             + _dot(aggr, wn1a_ref[...])
             + bn1_ref[...])
        u = _silu(z)
        u = _silu(_dot(u, wn2_ref[...]) + bn2_ref[...])
        mlp = _dot(u, wn3_ref[...]) + bn3_ref[...]
        h_new = hb + mlp
        if lnorm is not None:
            mu = jnp.mean(h_new, axis=-1, keepdims=True)
            var = jnp.mean((h_new - mu) ** 2, axis=-1, keepdims=True)
            h_new = (h_new - mu) / jnp.sqrt(var + 1e-5) * g_ref[...] + gb_ref[...]
            h_new = hb + h_new
        h_out_ref[...] = h_new
        if with_coord:
            xp_out_ref[...] = xp_ref[...] + px0_ref[0] + px1_ref[0]
        if fin is not None:
            fin_ref[...] = _dot(h_new, wo_ref[...]) + bo_ref[...]
        else:
            hr_ref[...] = _dot(h_new, w1r_ref[...]) + b1_ref[...]
            hc_ref[...] = _dot(h_new, w1c_ref[...])

    in_specs = [blk(TC_NODE_BLK, d), blk3(TC_NODE_BLK, d), blk3(TC_NODE_BLK, d)]
    args = [h, pd[:1], pd[1:]]
    if with_coord:
        in_specs += [blk(TC_NODE_BLK, XPAD), blk3(TC_NODE_BLK, XPAD),
                     blk3(TC_NODE_BLK, XPAD)]
        args += [xpad, px[:1], px[1:]]
    in_specs += [full(d, d), full(d, d), full(1, d), full(d, d), full(1, d),
                 full(d, d), full(1, d)]
    args += list(nwts)
    if lnorm is not None:
        in_specs += [full(1, d), full(1, d)]
        args += list(lnorm)
    if fin is not None:
        od = fin[0].shape[1]
        in_specs += [full(d, od), full(1, od)]
        args += list(fin)
    else:
        in_specs += [full(d, d), full(1, d), full(d, d)]
        args += list(nxt)

    out_specs = [blk(TC_NODE_BLK, d)]
    out_shape = [jax.ShapeDtypeStruct((n, d), F32)]
    if with_coord:
        out_specs.append(blk(TC_NODE_BLK, XPAD))
        out_shape.append(jax.ShapeDtypeStruct((n, XPAD), F32))
    if fin is not None:
        od = fin[0].shape[1]
        out_specs.append(blk(TC_NODE_BLK, od))
        out_shape.append(jax.ShapeDtypeStruct((n, od), F32))
    else:
        out_specs += [blk(TC_NODE_BLK, d), blk(TC_NODE_BLK, d)]
        out_shape += [jax.ShapeDtypeStruct((n, d), F32)] * 2

    res = pl.pallas_call(
        body, grid=grid, in_specs=in_specs, out_specs=out_specs,
        out_shape=out_shape,
    )(*args)

    it = iter(res)
    h_out = next(it)
    xp_out = next(it) if with_coord else None
    rest = tuple(it)
    return h_out, xp_out, rest


def _edge_weights(p, d, ed):
    (w1, b1), (w2, b2), (w3, b3) = p["edge"]
    wea = w1[:ed]
    w1r = w1[ed:ed + d]
    w1c = w1[ed + d:ed + 2 * d]
    w1d = w1[ed + 2 * d:ed + 2 * d + 1]
    (wa1, ba1), (wa2, ba2) = p["att"]
    (wc1, bc1), (wc2, bc2), (wc3, bc3) = p["coord"]
    cd = wc3.shape[1]
    wc3p = jnp.pad(wc3, ((0, 0), (0, XPAD - cd)))
    bc3p = jnp.pad(bc3.reshape(1, cd), ((0, 0), (0, XPAD - cd)))
    r2 = lambda b: b.reshape(1, -1)
    wts = (wea, w1d, w2, r2(b2), w3, r2(b3), wa1, r2(ba1),
           wa2.reshape(1, d), jnp.broadcast_to(ba2.reshape(1, 1), (1, d)),
           wc1, r2(bc1), wc2, r2(bc2), wc3p, bc3p)
    return wts, (w1r, r2(b1), w1c)


def kernel(h, x, edge_index, edge_attr, params):
    n, d = h.shape
    cd = x.shape[1]
    e = edge_index.shape[1]
    row = edge_index[0].astype(jnp.int32)
    col = edge_index[1].astype(jnp.int32)
    row2d = row.reshape(1, e)
    col2d = col.reshape(1, e)
    xpad = jnp.pad(x.astype(F32), ((0, 0), (0, XPAD - cd)))
    z_d = jnp.zeros((2, n, d), F32)
    z_x = jnp.zeros((2, n, XPAD), F32)

    layers = params["layers"]
    nlayers = len(layers)
    ed = edge_attr.shape[1]
    ewts = []
    projs = []
    for p in layers:
        w, pr = _edge_weights(p, d, ed)
        ewts.append(w)
        projs.append(pr)

    hr_tab, hc_tab = _tc_project(h, *projs[0])
    out = None
    for l, p in enumerate(layers):
        last = l == nlayers - 1
        hr_g, hc_g = _sc_gather_pair(hr_tab, hc_tab, row2d, col2d,
                                     untiled=False)
        xr_g, xc_g = _sc_gather_pair(xpad, xpad, row2d, col2d, untiled=True)
        eo, cw = _tc_edge(hr_g, hc_g, xr_g, xc_g, edge_attr, ewts[l],
                          with_coord=not last)
        pd = _sc_scatter(eo, row, z_d, untiled=False)
        px = _sc_scatter(cw, row, z_x, untiled=True) if not last else None
        (wn1, bn1), (wn2, bn2), (wn3, bn3) = p["node"]
        nwts = (wn1[:d], wn1[d:], bn1.reshape(1, -1), wn2,
                bn2.reshape(1, -1), wn3, bn3.reshape(1, -1))
        lnorm = None
        if l > 0:
            g, gb = p["ln"]
            lnorm = (g.reshape(1, -1), gb.reshape(1, -1))
        if last:
            wo, bo = params["out"]
            fin = (wo, bo.reshape(1, -1))
            h, xpad, rest = _tc_node(h, None, pd, None, nwts, lnorm, None, fin)
            out = rest[0]
        else:
            h, xpad, rest = _tc_node(h, xpad, pd, px, nwts, lnorm,
                                     projs[l + 1], None)
            hr_tab, hc_tab = rest
    return out
